# baseline (speedup 1.0000x reference)
import functools

import jax
import jax.numpy as jnp
from jax import lax
from jax.experimental import pallas as pl
from jax.experimental.pallas import tpu as pltpu, tpu_sc as plsc

BS = 2
N = 8192
M = 4096
C = 128
BM = 256
BIG = 1e30

NC = 2
NS = 16
NW = NC * NS
CPW = M // NW


def _knn_body(b, child_ref, xyz_ref, idx_ref, w_ref):
    c = child_ref[0]
    p = xyz_ref[0]
    t0 = c[:, 0:1] - p[0:1, :]
    t1 = c[:, 1:2] - p[1:2, :]
    t2 = c[:, 2:3] - p[2:3, :]
    d2 = (t0 * t0 + t1 * t1) + t2 * t2

    iota = lax.broadcasted_iota(jnp.int32, (BM, N), 1).astype(jnp.float32)
    fn = jnp.float32(N)

    m1 = jnp.min(d2, axis=1, keepdims=True)
    a1 = jnp.min(jnp.where(d2 == m1, iota, fn), axis=1, keepdims=True)
    d2b = jnp.where(iota == a1, BIG, d2)
    m2 = jnp.min(d2b, axis=1, keepdims=True)
    a2 = jnp.min(jnp.where(d2b == m2, iota, fn), axis=1, keepdims=True)
    d2c = jnp.where(iota == a2, BIG, d2b)
    m3 = jnp.min(d2c, axis=1, keepdims=True)
    a3 = jnp.min(jnp.where(d2c == m3, iota, fn), axis=1, keepdims=True)

    d = jnp.sqrt(jnp.maximum(jnp.concatenate([m1, m2, m3], axis=1), 1e-12))
    inv = 1.0 / (d + 1e-8)
    w = inv / jnp.sum(inv, axis=1, keepdims=True)

    ai = jnp.concatenate([a1, a2, a3], axis=1).astype(jnp.int32)
    idx_ref[0] = jnp.transpose(ai, (1, 0)) + b * N
    w_ref[0] = jnp.transpose(w, (1, 0))


def _three_nn_tc(child_xyz, xyz_t, b):
    grid = (M // BM,)
    return pl.pallas_call(
        functools.partial(_knn_body, b),
        grid=grid,
        in_specs=[
            pl.BlockSpec((1, BM, 3), lambda i: (b, i, 0)),
            pl.BlockSpec((1, 3, N), lambda i: (b, 0, 0)),
        ],
        out_specs=[
            pl.BlockSpec((1, 3, BM), lambda i: (0, 0, i)),
            pl.BlockSpec((1, 3, BM), lambda i: (0, 0, i)),
        ],
        out_shape=[
            jax.ShapeDtypeStruct((1, 3, M), jnp.int32),
            jax.ShapeDtypeStruct((1, 3, M), jnp.float32),
        ],
    )(child_xyz, xyz_t)


def _gather_body(feats_hbm, idx0_hbm, idx1_hbm, idx2_hbm,
                 w0_hbm, w1_hbm, w2_hbm, out_hbm,
                 idx0_v, idx1_v, idx2_v, w0_v, w1_v, w2_v,
                 rows_v, out_v, sem):
    wid = lax.axis_index("s") * NC + lax.axis_index("c")
    base = wid * CPW

    sl = pl.ds(base, CPW)
    pltpu.sync_copy(idx0_hbm.at[sl], idx0_v)
    pltpu.sync_copy(idx1_hbm.at[sl], idx1_v)
    pltpu.sync_copy(idx2_hbm.at[sl], idx2_v)
    pltpu.sync_copy(w0_hbm.at[sl], w0_v.at[pl.ds(0, CPW)])
    pltpu.sync_copy(w1_hbm.at[sl], w1_v.at[pl.ds(0, CPW)])
    pltpu.sync_copy(w2_hbm.at[sl], w2_v.at[pl.ds(0, CPW)])

    c0 = pltpu.async_copy(feats_hbm.at[idx0_v], rows_v.at[pl.ds(0, CPW)], sem)
    c1 = pltpu.async_copy(feats_hbm.at[idx1_v], rows_v.at[pl.ds(CPW, CPW)], sem)
    c2 = pltpu.async_copy(feats_hbm.at[idx2_v], rows_v.at[pl.ds(2 * CPW, CPW)],
                          sem)
    c0.wait()
    c1.wait()
    c2.wait()

    def child_body(lc, _):
        w0 = jnp.full((16,), w0_v[pl.ds(lc, 16)][0], jnp.float32)
        w1 = jnp.full((16,), w1_v[pl.ds(lc, 16)][0], jnp.float32)
        w2 = jnp.full((16,), w2_v[pl.ds(lc, 16)][0], jnp.float32)
        for dc in range(C // 16):
            s = pl.ds(dc * 16, 16)
            acc = (w0 * rows_v[lc, s] + w1 * rows_v[CPW + lc, s]
                   + w2 * rows_v[2 * CPW + lc, s])
            out_v[lc, s] = acc
        return ()

    lax.fori_loop(0, CPW, child_body, (), unroll=False)
    pltpu.sync_copy(out_v, out_hbm.at[pl.ds(base, CPW)])


def _gather_sc(feats_flat, idx, w):
    mesh = plsc.VectorSubcoreMesh(core_axis_name="c", subcore_axis_name="s",
                                  num_cores=NC, num_subcores=NS)
    f = pl.kernel(
        _gather_body,
        out_type=jax.ShapeDtypeStruct((M, C), jnp.float32),
        mesh=mesh,
        scratch_types=[
            pltpu.VMEM((CPW,), jnp.int32),
            pltpu.VMEM((CPW,), jnp.int32),
            pltpu.VMEM((CPW,), jnp.int32),
            pltpu.VMEM((CPW + 16,), jnp.float32),
            pltpu.VMEM((CPW + 16,), jnp.float32),
            pltpu.VMEM((CPW + 16,), jnp.float32),
            pltpu.VMEM((3 * CPW, C), jnp.float32),
            pltpu.VMEM((CPW, C), jnp.float32),
            pltpu.SemaphoreType.DMA,
        ],
    )
    return f(feats_flat, idx[0], idx[1], idx[2], w[0], w[1], w[2])


@jax.jit
def kernel(xyz, feats, child_xyz):
    xyz_t = jnp.transpose(xyz, (0, 2, 1))
    feats_flat = jnp.transpose(feats, (0, 2, 1)).reshape(BS * N, C)
    outs = []
    for b in range(BS):
        idx, w = _three_nn_tc(child_xyz, xyz_t, b)
        outs.append(_gather_sc(feats_flat, idx[0], w[0]))
    out = jnp.stack(outs)
    child_feats = jnp.transpose(out, (0, 2, 1))
    return (child_xyz, child_feats)


# speedup vs baseline: 1.0080x; 1.0080x over previous
import functools

import jax
import jax.numpy as jnp
from jax import lax
from jax.experimental import pallas as pl
from jax.experimental.pallas import tpu as pltpu, tpu_sc as plsc

BS = 2
N = 8192
M = 4096
C = 128
BM = 256
BIG = 1e30

NC = 2
NS = 16
NW = NC * NS
CPW = M // NW


def _knn_body(b, child_ref, xyz_ref, idx_ref, w_ref):
    c = child_ref[0]
    p = xyz_ref[0]
    t0 = c[:, 0:1] - p[0:1, :]
    t1 = c[:, 1:2] - p[1:2, :]
    t2 = c[:, 2:3] - p[2:3, :]
    d2 = (t0 * t0 + t1 * t1) + t2 * t2

    iota = lax.broadcasted_iota(jnp.int32, (BM, N), 1).astype(jnp.float32)
    fn = jnp.float32(N)

    m1 = jnp.min(d2, axis=1, keepdims=True)
    a1 = jnp.min(jnp.where(d2 == m1, iota, fn), axis=1, keepdims=True)
    d2b = jnp.where(iota == a1, BIG, d2)
    m2 = jnp.min(d2b, axis=1, keepdims=True)
    a2 = jnp.min(jnp.where(d2b == m2, iota, fn), axis=1, keepdims=True)
    d2c = jnp.where(iota == a2, BIG, d2b)
    m3 = jnp.min(d2c, axis=1, keepdims=True)
    a3 = jnp.min(jnp.where(d2c == m3, iota, fn), axis=1, keepdims=True)

    d = jnp.sqrt(jnp.maximum(jnp.concatenate([m1, m2, m3], axis=1), 1e-12))
    inv = 1.0 / (d + 1e-8)
    w = inv / jnp.sum(inv, axis=1, keepdims=True)

    ai = jnp.concatenate([a1, a2, a3], axis=1).astype(jnp.int32)
    idx_ref[0] = jnp.transpose(ai, (1, 0)) + b * N
    w_ref[0] = jnp.transpose(w, (1, 0))


def _three_nn_tc(child_xyz, xyz_t, b):
    grid = (M // BM,)
    return pl.pallas_call(
        functools.partial(_knn_body, b),
        grid=grid,
        in_specs=[
            pl.BlockSpec((1, BM, 3), lambda i: (b, i, 0)),
            pl.BlockSpec((1, 3, N), lambda i: (b, 0, 0)),
        ],
        out_specs=[
            pl.BlockSpec((1, 3, BM), lambda i: (0, 0, i)),
            pl.BlockSpec((1, 3, BM), lambda i: (0, 0, i)),
        ],
        out_shape=[
            jax.ShapeDtypeStruct((1, 3, M), jnp.int32),
            jax.ShapeDtypeStruct((1, 3, M), jnp.float32),
        ],
    )(child_xyz, xyz_t)


def _gather_body(feats_hbm, idx0_hbm, idx1_hbm, idx2_hbm,
                 w0_hbm, w1_hbm, w2_hbm, out_hbm,
                 idx0_v, idx1_v, idx2_v, w0_v, w1_v, w2_v,
                 rows_v, out_v, sem):
    wid = lax.axis_index("s") * NC + lax.axis_index("c")
    base = wid * CPW

    sl = pl.ds(base, CPW)
    s0 = pltpu.async_copy(idx0_hbm.at[sl], idx0_v, sem)
    s1 = pltpu.async_copy(idx1_hbm.at[sl], idx1_v, sem)
    s2 = pltpu.async_copy(idx2_hbm.at[sl], idx2_v, sem)
    s3 = pltpu.async_copy(w0_hbm.at[sl], w0_v.at[pl.ds(0, CPW)], sem)
    s4 = pltpu.async_copy(w1_hbm.at[sl], w1_v.at[pl.ds(0, CPW)], sem)
    s5 = pltpu.async_copy(w2_hbm.at[sl], w2_v.at[pl.ds(0, CPW)], sem)
    s0.wait(); s1.wait(); s2.wait(); s3.wait(); s4.wait(); s5.wait()

    c0 = pltpu.async_copy(feats_hbm.at[idx0_v], rows_v.at[pl.ds(0, CPW)], sem)
    c1 = pltpu.async_copy(feats_hbm.at[idx1_v], rows_v.at[pl.ds(CPW, CPW)], sem)
    c2 = pltpu.async_copy(feats_hbm.at[idx2_v], rows_v.at[pl.ds(2 * CPW, CPW)],
                          sem)
    c0.wait()
    c1.wait()
    c2.wait()

    def child_body(lc, _):
        w0 = jnp.full((16,), w0_v[pl.ds(lc, 16)][0], jnp.float32)
        w1 = jnp.full((16,), w1_v[pl.ds(lc, 16)][0], jnp.float32)
        w2 = jnp.full((16,), w2_v[pl.ds(lc, 16)][0], jnp.float32)
        for dc in range(C // 16):
            s = pl.ds(dc * 16, 16)
            acc = (w0 * rows_v[lc, s] + w1 * rows_v[CPW + lc, s]
                   + w2 * rows_v[2 * CPW + lc, s])
            out_v[lc, s] = acc
        return ()

    lax.fori_loop(0, CPW, child_body, (), unroll=False)
    pltpu.sync_copy(out_v, out_hbm.at[pl.ds(base, CPW)])


def _gather_sc(feats_flat, idx, w):
    mesh = plsc.VectorSubcoreMesh(core_axis_name="c", subcore_axis_name="s",
                                  num_cores=NC, num_subcores=NS)
    f = pl.kernel(
        _gather_body,
        out_type=jax.ShapeDtypeStruct((M, C), jnp.float32),
        mesh=mesh,
        scratch_types=[
            pltpu.VMEM((CPW,), jnp.int32),
            pltpu.VMEM((CPW,), jnp.int32),
            pltpu.VMEM((CPW,), jnp.int32),
            pltpu.VMEM((CPW + 16,), jnp.float32),
            pltpu.VMEM((CPW + 16,), jnp.float32),
            pltpu.VMEM((CPW + 16,), jnp.float32),
            pltpu.VMEM((3 * CPW, C), jnp.float32),
            pltpu.VMEM((CPW, C), jnp.float32),
            pltpu.SemaphoreType.DMA,
        ],
    )
    return f(feats_flat, idx[0], idx[1], idx[2], w[0], w[1], w[2])


@jax.jit
def kernel(xyz, feats, child_xyz):
    xyz_t = jnp.transpose(xyz, (0, 2, 1))
    feats_flat = jnp.transpose(feats, (0, 2, 1)).reshape(BS * N, C)
    outs = []
    for b in range(BS):
        idx, w = _three_nn_tc(child_xyz, xyz_t, b)
        outs.append(_gather_sc(feats_flat, idx[0], w[0]))
    out = jnp.stack(outs)
    child_feats = jnp.transpose(out, (0, 2, 1))
    return (child_xyz, child_feats)
